# baseline (speedup 1.0000x reference)
import jax
import jax.numpy as jnp
from jax.experimental import pallas as pl
from jax.experimental.pallas import tpu as pltpu

D = 768
E = 8
H = 12
DH = D // H
HID = 2 * D
N = 2048
T = N + E
_CH = 256
_SQRT2 = 1.4142135623730951


def _gelu_exact(x):
    return x * 0.5 * (1.0 + jax.lax.erf(x / _SQRT2))


def _w_copies(W1h, W2h, W3h, w1_s, w2_s, w3_s, sems, e, slot):
    cs = []
    for j in (0, 1):
        cs.append(pltpu.make_async_copy(W1h.at[e, j], w1_s.at[slot, j],
                                        sems.at[slot, j]))
        cs.append(pltpu.make_async_copy(W2h.at[e, j], w2_s.at[slot, j],
                                        sems.at[slot, 2 + j]))
        cs.append(pltpu.make_async_copy(W3h.at[e, j], w3_s.at[slot, j],
                                        sems.at[slot, 4 + j]))
    return cs


def _fused_kernel(et_ref, xh, ln1g, ln1b, ln2g, ln2b, Wqh, Wkvh, Wprojh, bproj,
                  moe_tok, W1h, W2h, W3h, b1_ref, b2_ref, b3_ref, Wg_row, bg,
                  ef_out, conf_out, fused_out,
                  kv_s, ao_s, q_s, x_s, wq_s, wkv_s, wproj_s,
                  w1_s, w2_s, w3_s, sems, isems):
    in_copies = [
        pltpu.make_async_copy(xh, x_s, isems.at[0]),
        pltpu.make_async_copy(Wkvh, wkv_s, isems.at[1]),
        pltpu.make_async_copy(Wqh, wq_s, isems.at[2]),
        pltpu.make_async_copy(Wprojh, wproj_s, isems.at[3]),
    ]
    for c in in_copies:
        c.start()
    for c in _w_copies(W1h, W2h, W3h, w1_s, w2_s, w3_s, sems, 0, 0):
        c.start()
    for c in _w_copies(W1h, W2h, W3h, w1_s, w2_s, w3_s, sems, 1, 1):
        c.start()

    ln1g_v = ln1g[...]
    ln1b_v = ln1b[...]
    in_copies[0].wait()
    in_copies[1].wait()
    x_ref = x_s
    wkv = wkv_s[...]

    def _ln1(x):
        m = jnp.mean(x, axis=1, keepdims=True)
        v = jnp.mean((x - m) ** 2, axis=1, keepdims=True)
        return (x - m) * jax.lax.rsqrt(v + 1e-5) * ln1g_v + ln1b_v

    xn8 = _ln1(et_ref[...])
    kv_s[:E, :] = jnp.dot(xn8, wkv,
                          preferred_element_type=jnp.float32).astype(jnp.bfloat16)

    def _kv_body(i, carry):
        xn_c = _ln1(x_ref[pl.ds(i * _CH, _CH), :])
        kv_s[pl.ds(E + i * _CH, _CH), :] = jnp.dot(
            xn_c, wkv, preferred_element_type=jnp.float32).astype(jnp.bfloat16)
        return carry

    jax.lax.fori_loop(0, N // _CH, _kv_body, 0)

    in_copies[2].wait()
    q_s[...] = jnp.dot(xn8, wq_s[...], preferred_element_type=jnp.float32)
    scale = DH ** -0.5

    for h in range(H):
        k_h = kv_s[:, h * DH:(h + 1) * DH].astype(jnp.float32)
        v_h = kv_s[:, D + h * DH:D + (h + 1) * DH].astype(jnp.float32)
        q_h = q_s[:, h * DH:(h + 1) * DH]
        s = jax.lax.dot_general(q_h, k_h, (((1,), (1,)), ((), ())),
                                preferred_element_type=jnp.float32) * scale
        s = s - jnp.max(s, axis=1, keepdims=True)
        p = jnp.exp(s)
        p = p / jnp.sum(p, axis=1, keepdims=True)
        o_h = jnp.dot(p, v_h, preferred_element_type=jnp.float32)
        ao_s[:, h * DH:(h + 1) * DH] = o_h
    in_copies[3].wait()
    ao = jnp.dot(ao_s[...], wproj_s[...], preferred_element_type=jnp.float32) + bproj[...]
    xc8 = et_ref[...] + ao

    m2 = jnp.mean(xc8, axis=1, keepdims=True)
    v2 = jnp.mean((xc8 - m2) ** 2, axis=1, keepdims=True)
    xn2 = (xc8 - m2) * jax.lax.rsqrt(v2 + 1e-5) * ln2g[...] + ln2b[...]

    scores = jax.lax.dot_general(xn2, moe_tok[...], (((1,), (1,)), ((), ())),
                                 preferred_element_type=jnp.float32)
    col = jax.lax.broadcasted_iota(jnp.int32, (E, E), 1)
    m1 = jnp.max(scores, axis=1, keepdims=True)
    idx1 = jnp.min(jnp.where(scores == m1, col, E), axis=1, keepdims=True)
    s2 = jnp.where(col == idx1, -jnp.inf, scores)
    m2s = jnp.max(s2, axis=1, keepdims=True)
    idx2 = jnp.min(jnp.where(s2 == m2s, col, E), axis=1, keepdims=True)

    acc = jnp.zeros((E, D), jnp.float32)
    for e in range(E):
        slot = e % 2
        for c in _w_copies(W1h, W2h, W3h, w1_s, w2_s, w3_s, sems, e, slot):
            c.wait()
        h1 = _gelu_exact(
            jnp.dot(xn2[:, :D // 2], w1_s[slot, 0], preferred_element_type=jnp.float32)
            + jnp.dot(xn2[:, D // 2:], w1_s[slot, 1], preferred_element_type=jnp.float32)
            + b1_ref[e:e + 1, :])
        h2 = _gelu_exact(
            jnp.dot(h1[:, :HID // 2], w2_s[slot, 0], preferred_element_type=jnp.float32)
            + jnp.dot(h1[:, HID // 2:], w2_s[slot, 1], preferred_element_type=jnp.float32)
            + b2_ref[e:e + 1, :])
        h3 = (jnp.dot(h2[:, :HID // 2], w3_s[slot, 0], preferred_element_type=jnp.float32)
              + jnp.dot(h2[:, HID // 2:], w3_s[slot, 1], preferred_element_type=jnp.float32)
              + b3_ref[e:e + 1, :])
        mask = 0.5 * ((idx1 == e).astype(jnp.float32)
                      + (idx2 == e).astype(jnp.float32))
        acc = acc + h3 * mask
        if e + 2 < E:
            for c in _w_copies(W1h, W2h, W3h, w1_s, w2_s, w3_s, sems, e + 2, slot):
                c.start()

    ef = xc8 + acc
    ef_out[...] = ef
    logit = jnp.sum(ef * Wg_row[...], axis=1, keepdims=True) + bg[0, 0]
    conf_out[...] = jnp.broadcast_to(jax.nn.sigmoid(logit), (E, 128))
    fused_out[...] = 0.5 * (x_ref[0:1, :] + x_ref[1:2, :])


def kernel(inputs, expert_tokens_outer, ln1_g, ln1_b, ln2_g, ln2_b, Wq, Wkv,
           Wproj, bproj, moe_tokens, W1, b1, W2, b2, W3, b3, Wg, bg):
    vmem = pl.BlockSpec(memory_space=pltpu.MemorySpace.VMEM)
    hbm = pl.BlockSpec(memory_space=pltpu.MemorySpace.HBM)

    ef, conf, fused = pl.pallas_call(
        _fused_kernel,
        in_specs=[vmem, hbm, vmem, vmem, vmem, vmem, hbm, hbm, hbm, vmem,
                  vmem, hbm, hbm, hbm] + [vmem] * 5,
        out_specs=[vmem, vmem, vmem],
        out_shape=[
            jax.ShapeDtypeStruct((E, D), jnp.float32),
            jax.ShapeDtypeStruct((E, 128), jnp.float32),
            jax.ShapeDtypeStruct((1, D), jnp.float32),
        ],
        scratch_shapes=[
            pltpu.VMEM((T, 2 * D), jnp.bfloat16),
            pltpu.VMEM((E, D), jnp.float32),
            pltpu.VMEM((E, D), jnp.float32),
            pltpu.VMEM((N, D), jnp.float32),
            pltpu.VMEM((D, D), jnp.float32),
            pltpu.VMEM((D, 2 * D), jnp.float32),
            pltpu.VMEM((D, D), jnp.float32),
            pltpu.VMEM((2, 2, D // 2, HID), jnp.float32),
            pltpu.VMEM((2, 2, HID // 2, HID), jnp.float32),
            pltpu.VMEM((2, 2, HID // 2, D), jnp.float32),
            pltpu.SemaphoreType.DMA((2, 6)),
            pltpu.SemaphoreType.DMA((4,)),
        ],
        compiler_params=pltpu.CompilerParams(
            vmem_limit_bytes=128 * 1024 * 1024,
        ),
    )(
        expert_tokens_outer, inputs[0],
        ln1_g.reshape(1, D), ln1_b.reshape(1, D),
        ln2_g.reshape(1, D), ln2_b.reshape(1, D),
        Wq, Wkv, Wproj, bproj.reshape(1, D),
        moe_tokens,
        W1.reshape(E, 2, D // 2, HID),
        W2.reshape(E, 2, HID // 2, HID),
        W3.reshape(E, 2, HID // 2, D),
        b1, b2, b3,
        Wg.reshape(1, D), bg.reshape(1, 1),
    )
    expert_features = ef.reshape(1, E, D)
    confidence = conf[:, :1].reshape(1, E, 1)
    return (expert_features, confidence, fused.reshape(1, D))


# speedup vs baseline: 1.0641x; 1.0641x over previous
import jax
import jax.numpy as jnp
from jax.experimental import pallas as pl
from jax.experimental.pallas import tpu as pltpu

D = 768
E = 8
H = 12
DH = D // H
HID = 2 * D
N = 2048
T = N + E
_CH = 256
_SQRT2 = 1.4142135623730951


def _gelu_exact(x):
    return x * 0.5 * (1.0 + jax.lax.erf(x / _SQRT2))


def _w_copies(W1h, W2h, W3h, w1_s, w2_s, w3_s, sems, e, slot):
    cs = []
    for j in (0, 1):
        cs.append(pltpu.make_async_copy(W1h.at[e, j], w1_s.at[slot, j],
                                        sems.at[slot, j]))
        cs.append(pltpu.make_async_copy(W2h.at[e, j], w2_s.at[slot, j],
                                        sems.at[slot, 2 + j]))
        cs.append(pltpu.make_async_copy(W3h.at[e, j], w3_s.at[slot, j],
                                        sems.at[slot, 4 + j]))
    return cs


def _fused_kernel(et_ref, x_ref, ln1g, ln1b, ln2g, ln2b, Wq, Wkv, Wproj, bproj,
                  moe_tok, W1h, W2h, W3h, b1_ref, b2_ref, b3_ref, Wg_row, bg,
                  ef_out, conf_out, fused_out,
                  kv_s, ao_s, q_s, w1_s, w2_s, w3_s, sems):
    for c in _w_copies(W1h, W2h, W3h, w1_s, w2_s, w3_s, sems, 0, 0):
        c.start()
    for c in _w_copies(W1h, W2h, W3h, w1_s, w2_s, w3_s, sems, 1, 1):
        c.start()

    ln1g_v = ln1g[...]
    ln1b_v = ln1b[...]
    wkv16 = Wkv[...].astype(jnp.bfloat16)

    def _ln1(x):
        m = jnp.mean(x, axis=1, keepdims=True)
        v = jnp.mean((x - m) ** 2, axis=1, keepdims=True)
        return (x - m) * jax.lax.rsqrt(v + 1e-5) * ln1g_v + ln1b_v

    xn8 = _ln1(et_ref[...])
    kv_s[:E, :] = jnp.dot(xn8.astype(jnp.bfloat16), wkv16,
                          preferred_element_type=jnp.float32).astype(jnp.bfloat16)

    def _kv_body(i, carry):
        xn_c = _ln1(x_ref[pl.ds(i * _CH, _CH), :])
        kv_s[pl.ds(E + i * _CH, _CH), :] = jnp.dot(
            xn_c.astype(jnp.bfloat16), wkv16,
            preferred_element_type=jnp.float32).astype(jnp.bfloat16)
        return carry

    jax.lax.fori_loop(0, N // _CH, _kv_body, 0)

    q_s[...] = jnp.dot(xn8, Wq[...], preferred_element_type=jnp.float32)
    scale = DH ** -0.5

    for h in range(H):
        k_h = kv_s[:, h * DH:(h + 1) * DH].astype(jnp.float32)
        v_h = kv_s[:, D + h * DH:D + (h + 1) * DH].astype(jnp.float32)
        q_h = q_s[:, h * DH:(h + 1) * DH]
        s = jax.lax.dot_general(q_h, k_h, (((1,), (1,)), ((), ())),
                                preferred_element_type=jnp.float32) * scale
        s = s - jnp.max(s, axis=1, keepdims=True)
        p = jnp.exp(s)
        p = p / jnp.sum(p, axis=1, keepdims=True)
        o_h = jnp.dot(p, v_h, preferred_element_type=jnp.float32)
        ao_s[:, h * DH:(h + 1) * DH] = o_h
    ao = jnp.dot(ao_s[...], Wproj[...], preferred_element_type=jnp.float32) + bproj[...]
    xc8 = et_ref[...] + ao

    m2 = jnp.mean(xc8, axis=1, keepdims=True)
    v2 = jnp.mean((xc8 - m2) ** 2, axis=1, keepdims=True)
    xn2 = (xc8 - m2) * jax.lax.rsqrt(v2 + 1e-5) * ln2g[...] + ln2b[...]

    scores = jax.lax.dot_general(xn2, moe_tok[...], (((1,), (1,)), ((), ())),
                                 preferred_element_type=jnp.float32)
    col = jax.lax.broadcasted_iota(jnp.int32, (E, E), 1)
    m1 = jnp.max(scores, axis=1, keepdims=True)
    idx1 = jnp.min(jnp.where(scores == m1, col, E), axis=1, keepdims=True)
    s2 = jnp.where(col == idx1, -jnp.inf, scores)
    m2s = jnp.max(s2, axis=1, keepdims=True)
    idx2 = jnp.min(jnp.where(s2 == m2s, col, E), axis=1, keepdims=True)

    acc = jnp.zeros((E, D), jnp.float32)
    for e in range(E):
        slot = e % 2
        for c in _w_copies(W1h, W2h, W3h, w1_s, w2_s, w3_s, sems, e, slot):
            c.wait()
        h1 = _gelu_exact(
            jnp.dot(xn2[:, :D // 2], w1_s[slot, 0], preferred_element_type=jnp.float32)
            + jnp.dot(xn2[:, D // 2:], w1_s[slot, 1], preferred_element_type=jnp.float32)
            + b1_ref[e:e + 1, :])
        h2 = _gelu_exact(
            jnp.dot(h1[:, :HID // 2], w2_s[slot, 0], preferred_element_type=jnp.float32)
            + jnp.dot(h1[:, HID // 2:], w2_s[slot, 1], preferred_element_type=jnp.float32)
            + b2_ref[e:e + 1, :])
        h3 = (jnp.dot(h2[:, :HID // 2], w3_s[slot, 0], preferred_element_type=jnp.float32)
              + jnp.dot(h2[:, HID // 2:], w3_s[slot, 1], preferred_element_type=jnp.float32)
              + b3_ref[e:e + 1, :])
        mask = 0.5 * ((idx1 == e).astype(jnp.float32)
                      + (idx2 == e).astype(jnp.float32))
        acc = acc + h3 * mask
        if e + 2 < E:
            for c in _w_copies(W1h, W2h, W3h, w1_s, w2_s, w3_s, sems, e + 2, slot):
                c.start()

    ef = xc8 + acc
    ef_out[...] = ef
    logit = jnp.sum(ef * Wg_row[...], axis=1, keepdims=True) + bg[0, 0]
    conf_out[...] = jnp.broadcast_to(jax.nn.sigmoid(logit), (E, 128))
    fused_out[...] = 0.5 * (x_ref[0:1, :] + x_ref[1:2, :])


def kernel(inputs, expert_tokens_outer, ln1_g, ln1_b, ln2_g, ln2_b, Wq, Wkv,
           Wproj, bproj, moe_tokens, W1, b1, W2, b2, W3, b3, Wg, bg):
    vmem = pl.BlockSpec(memory_space=pltpu.MemorySpace.VMEM)
    hbm = pl.BlockSpec(memory_space=pltpu.MemorySpace.HBM)

    ef, conf, fused = pl.pallas_call(
        _fused_kernel,
        in_specs=[vmem] * 11 + [hbm, hbm, hbm] + [vmem] * 5,
        out_specs=[vmem, vmem, vmem],
        out_shape=[
            jax.ShapeDtypeStruct((E, D), jnp.float32),
            jax.ShapeDtypeStruct((E, 128), jnp.float32),
            jax.ShapeDtypeStruct((1, D), jnp.float32),
        ],
        scratch_shapes=[
            pltpu.VMEM((T, 2 * D), jnp.bfloat16),
            pltpu.VMEM((E, D), jnp.float32),
            pltpu.VMEM((E, D), jnp.float32),
            pltpu.VMEM((2, 2, D // 2, HID), jnp.float32),
            pltpu.VMEM((2, 2, HID // 2, HID), jnp.float32),
            pltpu.VMEM((2, 2, HID // 2, D), jnp.float32),
            pltpu.SemaphoreType.DMA((2, 6)),
        ],
        compiler_params=pltpu.CompilerParams(
            vmem_limit_bytes=128 * 1024 * 1024,
        ),
    )(
        expert_tokens_outer, inputs[0],
        ln1_g.reshape(1, D), ln1_b.reshape(1, D),
        ln2_g.reshape(1, D), ln2_b.reshape(1, D),
        Wq, Wkv, Wproj, bproj.reshape(1, D),
        moe_tokens,
        W1.reshape(E, 2, D // 2, HID),
        W2.reshape(E, 2, HID // 2, HID),
        W3.reshape(E, 2, HID // 2, D),
        b1, b2, b3,
        Wg.reshape(1, D), bg.reshape(1, 1),
    )
    expert_features = ef.reshape(1, E, D)
    confidence = conf[:, :1].reshape(1, E, 1)
    return (expert_features, confidence, fused.reshape(1, D))


# speedup vs baseline: 1.1122x; 1.0452x over previous
import jax
import jax.numpy as jnp
from jax.experimental import pallas as pl
from jax.experimental.pallas import tpu as pltpu

D = 768
E = 8
H = 12
DH = D // H
HID = 2 * D
N = 2048
T = N + E
_CH = 256
_SQRT2 = 1.4142135623730951

_KA = 11
_KB = 4
_NA = E * 6
_NB = E * 2


def _gelu_exact(x):
    return x * 0.5 * (1.0 + jax.lax.erf(x / _SQRT2))


def _a_copy(W1h, W2h, wa_s, asems, ga):
    e, sub = divmod(ga, 6)
    src = W1h.at[e, sub] if sub < 2 else W2h.at[e, sub - 2]
    return pltpu.make_async_copy(src, wa_s.at[ga % _KA], asems.at[ga % _KA])


def _b_copy(W3h, wb_s, bsems, gb):
    e, j = divmod(gb, 2)
    return pltpu.make_async_copy(W3h.at[e, j], wb_s.at[gb % _KB],
                                 bsems.at[gb % _KB])


def _fused_kernel(et_ref, x_ref, ln1g, ln1b, ln2g, ln2b, Wq, Wkv, Wproj, bproj,
                  moe_tok, W1h, W2h, W3h, b1_ref, b2_ref, b3_ref, Wg_row, bg,
                  ef_out, conf_out, fused_out,
                  kv_s, ao_s, q_s, wa_s, wb_s, asems, bsems):
    for ga in range(_KA):
        _a_copy(W1h, W2h, wa_s, asems, ga).start()
    for gb in range(_KB):
        _b_copy(W3h, wb_s, bsems, gb).start()

    ln1g_v = ln1g[...]
    ln1b_v = ln1b[...]
    wkv16 = Wkv[...].astype(jnp.bfloat16)

    def _ln1(x):
        m = jnp.mean(x, axis=1, keepdims=True)
        v = jnp.mean((x - m) ** 2, axis=1, keepdims=True)
        return (x - m) * jax.lax.rsqrt(v + 1e-5) * ln1g_v + ln1b_v

    xn8 = _ln1(et_ref[...])
    kv_s[:E, :] = jnp.dot(xn8.astype(jnp.bfloat16), wkv16,
                          preferred_element_type=jnp.float32).astype(jnp.bfloat16)

    def _kv_body(i, carry):
        xn_c = _ln1(x_ref[pl.ds(i * _CH, _CH), :])
        kv_s[pl.ds(E + i * _CH, _CH), :] = jnp.dot(
            xn_c.astype(jnp.bfloat16), wkv16,
            preferred_element_type=jnp.float32).astype(jnp.bfloat16)
        return carry

    jax.lax.fori_loop(0, N // _CH, _kv_body, 0)

    q_s[...] = jnp.dot(xn8, Wq[...], preferred_element_type=jnp.float32)
    scale = DH ** -0.5

    for h in range(H):
        k_h = kv_s[:, h * DH:(h + 1) * DH].astype(jnp.float32)
        v_h = kv_s[:, D + h * DH:D + (h + 1) * DH].astype(jnp.float32)
        q_h = q_s[:, h * DH:(h + 1) * DH]
        s = jax.lax.dot_general(q_h, k_h, (((1,), (1,)), ((), ())),
                                preferred_element_type=jnp.float32) * scale
        s = s - jnp.max(s, axis=1, keepdims=True)
        p = jnp.exp(s)
        p = p / jnp.sum(p, axis=1, keepdims=True)
        o_h = jnp.dot(p, v_h, preferred_element_type=jnp.float32)
        ao_s[:, h * DH:(h + 1) * DH] = o_h
    ao = jnp.dot(ao_s[...], Wproj[...], preferred_element_type=jnp.float32) + bproj[...]
    xc8 = et_ref[...] + ao

    m2 = jnp.mean(xc8, axis=1, keepdims=True)
    v2 = jnp.mean((xc8 - m2) ** 2, axis=1, keepdims=True)
    xn2 = (xc8 - m2) * jax.lax.rsqrt(v2 + 1e-5) * ln2g[...] + ln2b[...]

    scores = jax.lax.dot_general(xn2, moe_tok[...], (((1,), (1,)), ((), ())),
                                 preferred_element_type=jnp.float32)
    col = jax.lax.broadcasted_iota(jnp.int32, (E, E), 1)
    m1 = jnp.max(scores, axis=1, keepdims=True)
    idx1 = jnp.min(jnp.where(scores == m1, col, E), axis=1, keepdims=True)
    s2 = jnp.where(col == idx1, -jnp.inf, scores)
    m2s = jnp.max(s2, axis=1, keepdims=True)
    idx2 = jnp.min(jnp.where(s2 == m2s, col, E), axis=1, keepdims=True)

    Q = HID // 4

    def _use_a(ga, lhs):
        c = _a_copy(W1h, W2h, wa_s, asems, ga)
        c.wait()
        out = jnp.dot(lhs, wa_s[ga % _KA], preferred_element_type=jnp.float32)
        if ga + _KA < _NA:
            _a_copy(W1h, W2h, wa_s, asems, ga + _KA).start()
        return out

    def _use_b(gb, lhs):
        c = _b_copy(W3h, wb_s, bsems, gb)
        c.wait()
        out = jnp.dot(lhs, wb_s[gb % _KB], preferred_element_type=jnp.float32)
        if gb + _KB < _NB:
            _b_copy(W3h, wb_s, bsems, gb + _KB).start()
        return out

    acc = jnp.zeros((E, D), jnp.float32)
    for e in range(E):
        ga0 = e * 6
        gb0 = e * 2
        p1 = _use_a(ga0, xn2[:, :Q]) + _use_a(ga0 + 1, xn2[:, Q:])
        h1 = _gelu_exact(p1 + b1_ref[e:e + 1, :])
        p2 = (_use_a(ga0 + 2, h1[:, :Q]) + _use_a(ga0 + 3, h1[:, Q:2 * Q])
              + _use_a(ga0 + 4, h1[:, 2 * Q:3 * Q]) + _use_a(ga0 + 5, h1[:, 3 * Q:]))
        h2 = _gelu_exact(p2 + b2_ref[e:e + 1, :])
        h3 = _use_b(gb0, h2[:, :D]) + _use_b(gb0 + 1, h2[:, D:]) + b3_ref[e:e + 1, :]
        mask = 0.5 * ((idx1 == e).astype(jnp.float32)
                      + (idx2 == e).astype(jnp.float32))
        acc = acc + h3 * mask

    ef = xc8 + acc
    ef_out[...] = ef
    logit = jnp.sum(ef * Wg_row[...], axis=1, keepdims=True) + bg[0, 0]
    conf_out[...] = jnp.broadcast_to(jax.nn.sigmoid(logit), (E, 128))
    fused_out[...] = 0.5 * (x_ref[0:1, :] + x_ref[1:2, :])


def kernel(inputs, expert_tokens_outer, ln1_g, ln1_b, ln2_g, ln2_b, Wq, Wkv,
           Wproj, bproj, moe_tokens, W1, b1, W2, b2, W3, b3, Wg, bg):
    vmem = pl.BlockSpec(memory_space=pltpu.MemorySpace.VMEM)
    hbm = pl.BlockSpec(memory_space=pltpu.MemorySpace.HBM)

    ef, conf, fused = pl.pallas_call(
        _fused_kernel,
        in_specs=[vmem] * 11 + [hbm, hbm, hbm] + [vmem] * 5,
        out_specs=[vmem, vmem, vmem],
        out_shape=[
            jax.ShapeDtypeStruct((E, D), jnp.float32),
            jax.ShapeDtypeStruct((E, 128), jnp.float32),
            jax.ShapeDtypeStruct((1, D), jnp.float32),
        ],
        scratch_shapes=[
            pltpu.VMEM((T, 2 * D), jnp.bfloat16),
            pltpu.VMEM((E, D), jnp.float32),
            pltpu.VMEM((E, D), jnp.float32),
            pltpu.VMEM((_KA, HID // 4, HID), jnp.float32),
            pltpu.VMEM((_KB, D, D), jnp.float32),
            pltpu.SemaphoreType.DMA((_KA,)),
            pltpu.SemaphoreType.DMA((_KB,)),
        ],
        compiler_params=pltpu.CompilerParams(
            vmem_limit_bytes=128 * 1024 * 1024,
        ),
    )(
        expert_tokens_outer, inputs[0],
        ln1_g.reshape(1, D), ln1_b.reshape(1, D),
        ln2_g.reshape(1, D), ln2_b.reshape(1, D),
        Wq, Wkv, Wproj, bproj.reshape(1, D),
        moe_tokens,
        W1.reshape(E, 2, D // 2, HID),
        W2.reshape(E, 4, HID // 4, HID),
        W3.reshape(E, 2, HID // 2, D),
        b1, b2, b3,
        Wg.reshape(1, D), bg.reshape(1, 1),
    )
    expert_features = ef.reshape(1, E, D)
    confidence = conf[:, :1].reshape(1, E, 1)
    return (expert_features, confidence, fused.reshape(1, D))
